# baseline (speedup 1.0000x reference)
import functools

import jax
import jax.numpy as jnp
from jax.experimental import pallas as pl

N = 4096
D_IN = 128
D_HID = 64
BR = 512
BRB = 1024


def _dot1(a, b):
    return jax.lax.dot_general(
        a.astype(jnp.bfloat16),
        b.astype(jnp.bfloat16),
        (((1,), (0,)), ((), ())),
        preferred_element_type=jnp.float32,
    )


def _prep_kernel(x_ref, w_ref, v_ref, xw_ref, s_ref):
    xw = _dot1(x_ref[...], w_ref[...])
    xw_ref[...] = xw
    f1 = _dot1(xw, v_ref[0])
    f2 = _dot1(xw, v_ref[1])
    s_ref[...] = jax.nn.sigmoid(f1 + f2)


def _half_softmax(p, s):
    vals = p * s
    e = jnp.where(vals != 0, jnp.exp(vals), 0.0)
    tot = jnp.sum(e, axis=-1, keepdims=True)
    c = jnp.where(tot > 0, 0.5 / tot, 0.0)
    return e * c


def _pass_a_kernel(local_ref, long_ref, s_ref, xw_ref, att_ref, h_ref):
    s = s_ref[...]
    att = _half_softmax(long_ref[...], s) + _half_softmax(local_ref[...], s)
    att16 = att.astype(jnp.bfloat16)
    att_ref[...] = att16
    h_ref[...] = jax.lax.dot_general(
        att16,
        xw_ref[...].astype(jnp.bfloat16),
        (((1,), (0,)), ((), ())),
        preferred_element_type=jnp.float32,
    )


def _bn_kernel(h_ref, hn_ref):
    h = h_ref[...]
    mu = jnp.mean(h, axis=0, keepdims=True)
    var = jnp.mean((h - mu) ** 2, axis=0, keepdims=True)
    hn_ref[...] = (h - mu) / jnp.sqrt(var + 1e-6)


def _pass_b_kernel(att_ref, hn_ref, wt_ref, out_ref):
    dz = _dot1(att_ref[...], hn_ref[...])
    dz = _dot1(dz, wt_ref[...])
    out_ref[...] = jnp.where(dz > 0, dz, jnp.exp(jnp.minimum(dz, 0.0)) - 1.0)


@functools.partial(jax.jit, static_argnames=())
def kernel(local_patten, long_range_patten, X, W, V):
    xw, s = pl.pallas_call(
        _prep_kernel,
        out_shape=(
            jax.ShapeDtypeStruct((N, D_HID), jnp.float32),
            jax.ShapeDtypeStruct((N, 1), jnp.float32),
        ),
    )(X, W, V)

    grid = (N // BR,)
    att, h = pl.pallas_call(
        _pass_a_kernel,
        grid=grid,
        in_specs=[
            pl.BlockSpec((BR, N), lambda i: (i, 0)),
            pl.BlockSpec((BR, N), lambda i: (i, 0)),
            pl.BlockSpec((BR, 1), lambda i: (i, 0)),
            pl.BlockSpec((N, D_HID), lambda i: (0, 0)),
        ],
        out_specs=(
            pl.BlockSpec((BR, N), lambda i: (i, 0)),
            pl.BlockSpec((BR, D_HID), lambda i: (i, 0)),
        ),
        out_shape=(
            jax.ShapeDtypeStruct((N, N), jnp.bfloat16),
            jax.ShapeDtypeStruct((N, D_HID), jnp.float32),
        ),
    )(local_patten, long_range_patten, s, xw)

    hn = pl.pallas_call(
        _bn_kernel,
        out_shape=jax.ShapeDtypeStruct((N, D_HID), jnp.float32),
    )(h)

    out = pl.pallas_call(
        _pass_b_kernel,
        grid=(N // BRB,),
        in_specs=[
            pl.BlockSpec((BRB, N), lambda i: (i, 0)),
            pl.BlockSpec((N, D_HID), lambda i: (0, 0)),
            pl.BlockSpec((D_HID, D_IN), lambda i: (0, 0)),
        ],
        out_specs=pl.BlockSpec((BRB, D_IN), lambda i: (i, 0)),
        out_shape=jax.ShapeDtypeStruct((N, D_IN), jnp.float32),
    )(att, hn, W.T)

    return out


# speedup vs baseline: 1.1090x; 1.1090x over previous
import functools

import jax
import jax.numpy as jnp
from jax.experimental import pallas as pl
from jax.experimental.pallas import tpu as pltpu

N = 4096
D_IN = 128
D_HID = 64
BRA = 128
BRB = 512
NA = N // BRA
NB = N // BRB


def _dot1(a, b):
    return jax.lax.dot_general(
        a.astype(jnp.bfloat16),
        b.astype(jnp.bfloat16),
        (((1,), (0,)), ((), ())),
        preferred_element_type=jnp.float32,
    )


def _prep_kernel(x_ref, w_ref, v_ref, xw_ref, s_ref):
    xw = _dot1(x_ref[...], w_ref[...])
    xw_ref[...] = xw
    f1 = _dot1(xw, v_ref[0])
    f2 = _dot1(xw, v_ref[1])
    s_ref[...] = jax.nn.sigmoid(f1 + f2)


def _half_softmax(p, s):
    vals = p * s
    e = jnp.where(vals != 0, jnp.exp(vals), 0.0)
    tot = jnp.sum(e, axis=-1, keepdims=True)
    c = jnp.where(tot > 0, 0.5 / tot, 0.0)
    return e * c


def _mono_kernel(local_ref, long_ref, s_ref, xw_ref, wt_ref, out_ref,
                 att_scr, h_scr, hn_scr):
    i = pl.program_id(0)

    @pl.when(i < NA)
    def _softmax_phase():
        s = s_ref[...]
        att = _half_softmax(long_ref[...], s) + _half_softmax(local_ref[...], s)
        att16 = att.astype(jnp.bfloat16)
        att_scr[pl.ds(i * BRA, BRA), :] = att16
        h_scr[pl.ds(i * BRA, BRA), :] = jax.lax.dot_general(
            att16,
            xw_ref[...].astype(jnp.bfloat16),
            (((1,), (0,)), ((), ())),
            preferred_element_type=jnp.float32,
        )

    @pl.when(i == NA)
    def _bn_phase():
        h = h_scr[...]
        mu = jnp.mean(h, axis=0, keepdims=True)
        var = jnp.mean((h - mu) ** 2, axis=0, keepdims=True)
        hn_scr[...] = (h - mu) / jnp.sqrt(var + 1e-6)

    @pl.when(i >= NA)
    def _decoder_phase():
        j = i - NA
        att16 = att_scr[pl.ds(j * BRB, BRB), :]
        dz = jax.lax.dot_general(
            att16,
            hn_scr[...].astype(jnp.bfloat16),
            (((1,), (0,)), ((), ())),
            preferred_element_type=jnp.float32,
        )
        dz = _dot1(dz, wt_ref[...])
        out_ref[...] = jnp.where(dz > 0, dz, jnp.exp(jnp.minimum(dz, 0.0)) - 1.0)


@functools.partial(jax.jit, static_argnames=())
def kernel(local_patten, long_range_patten, X, W, V):
    xw, s = pl.pallas_call(
        _prep_kernel,
        out_shape=(
            jax.ShapeDtypeStruct((N, D_HID), jnp.float32),
            jax.ShapeDtypeStruct((N, 1), jnp.float32),
        ),
    )(X, W, V)

    out = pl.pallas_call(
        _mono_kernel,
        grid=(NA + NB,),
        in_specs=[
            pl.BlockSpec((BRA, N), lambda i: (jnp.minimum(i, NA - 1), 0)),
            pl.BlockSpec((BRA, N), lambda i: (jnp.minimum(i, NA - 1), 0)),
            pl.BlockSpec((BRA, 1), lambda i: (jnp.minimum(i, NA - 1), 0)),
            pl.BlockSpec((N, D_HID), lambda i: (0, 0)),
            pl.BlockSpec((D_HID, D_IN), lambda i: (0, 0)),
        ],
        out_specs=pl.BlockSpec(
            (BRB, D_IN), lambda i: (jnp.maximum(i - NA, 0), 0)
        ),
        out_shape=jax.ShapeDtypeStruct((N, D_IN), jnp.float32),
        scratch_shapes=[
            pltpu.VMEM((N, N), jnp.bfloat16),
            pltpu.VMEM((N, D_HID), jnp.float32),
            pltpu.VMEM((N, D_HID), jnp.float32),
        ],
    )(local_patten, long_range_patten, s, xw, W.T)

    return out


# speedup vs baseline: 1.2216x; 1.1015x over previous
import functools

import jax
import jax.numpy as jnp
from jax.experimental import pallas as pl
from jax.experimental.pallas import tpu as pltpu

N = 4096
D_IN = 128
D_HID = 64
BRA = 256
BRB = 1024
NA = N // BRA
NB = N // BRB


def _dot1(a, b):
    return jax.lax.dot_general(
        a.astype(jnp.bfloat16),
        b.astype(jnp.bfloat16),
        (((1,), (0,)), ((), ())),
        preferred_element_type=jnp.float32,
    )


def _prep_kernel(x_ref, w_ref, v_ref, xw_ref, s_ref):
    xw = _dot1(x_ref[...], w_ref[...])
    xw_ref[...] = xw
    f1 = _dot1(xw, v_ref[0])
    f2 = _dot1(xw, v_ref[1])
    s_ref[...] = jax.nn.sigmoid(f1 + f2)


def _half_softmax(p, s):
    vals = p * s
    e = jnp.where(vals != 0, jnp.exp(vals), 0.0)
    tot = jnp.sum(e, axis=-1, keepdims=True)
    c = jnp.where(tot > 0, 0.5 / tot, 0.0)
    return e * c


def _mono_kernel(local_ref, long_ref, s_ref, xw_ref, wt_ref, out_ref,
                 att_scr, h_scr, hn_scr):
    i = pl.program_id(0)

    @pl.when(i < NA)
    def _softmax_phase():
        s = s_ref[...]
        att = _half_softmax(long_ref[...], s) + _half_softmax(local_ref[...], s)
        att16 = att.astype(jnp.bfloat16)
        att_scr[pl.ds(i * BRA, BRA), :] = att16
        h_scr[pl.ds(i * BRA, BRA), :] = jax.lax.dot_general(
            att16,
            xw_ref[...].astype(jnp.bfloat16),
            (((1,), (0,)), ((), ())),
            preferred_element_type=jnp.float32,
        )

    @pl.when(i == NA)
    def _bn_phase():
        h = h_scr[...]
        mu = jnp.mean(h, axis=0, keepdims=True)
        var = jnp.mean((h - mu) ** 2, axis=0, keepdims=True)
        hn_scr[...] = (h - mu) / jnp.sqrt(var + 1e-6)

    @pl.when(i >= NA)
    def _decoder_phase():
        j = i - NA
        att16 = att_scr[pl.ds(j * BRB, BRB), :]
        dz = jax.lax.dot_general(
            att16,
            hn_scr[...].astype(jnp.bfloat16),
            (((1,), (0,)), ((), ())),
            preferred_element_type=jnp.float32,
        )
        dz = _dot1(dz, wt_ref[...])
        out_ref[...] = jnp.where(dz > 0, dz, jnp.exp(jnp.minimum(dz, 0.0)) - 1.0)


@functools.partial(jax.jit, static_argnames=())
def kernel(local_patten, long_range_patten, X, W, V):
    xw, s = pl.pallas_call(
        _prep_kernel,
        out_shape=(
            jax.ShapeDtypeStruct((N, D_HID), jnp.float32),
            jax.ShapeDtypeStruct((N, 1), jnp.float32),
        ),
    )(X, W, V)

    out = pl.pallas_call(
        _mono_kernel,
        grid=(NA + NB,),
        in_specs=[
            pl.BlockSpec((BRA, N), lambda i: (jnp.minimum(i, NA - 1), 0)),
            pl.BlockSpec((BRA, N), lambda i: (jnp.minimum(i, NA - 1), 0)),
            pl.BlockSpec((BRA, 1), lambda i: (jnp.minimum(i, NA - 1), 0)),
            pl.BlockSpec((N, D_HID), lambda i: (0, 0)),
            pl.BlockSpec((D_HID, D_IN), lambda i: (0, 0)),
        ],
        out_specs=pl.BlockSpec(
            (BRB, D_IN), lambda i: (jnp.maximum(i - NA, 0), 0)
        ),
        out_shape=jax.ShapeDtypeStruct((N, D_IN), jnp.float32),
        scratch_shapes=[
            pltpu.VMEM((N, N), jnp.bfloat16),
            pltpu.VMEM((N, D_HID), jnp.float32),
            pltpu.VMEM((N, D_HID), jnp.float32),
        ],
        compiler_params=pltpu.CompilerParams(vmem_limit_bytes=67000000),
    )(local_patten, long_range_patten, s, xw, W.T)

    return out


# speedup vs baseline: 1.2551x; 1.0275x over previous
import functools

import jax
import jax.numpy as jnp
from jax.experimental import pallas as pl
from jax.experimental.pallas import tpu as pltpu

N = 4096
D_IN = 128
D_HID = 64
BRA = 256
BRB = 1024
NA = N // BRA
NB = N // BRB


def _dot1(a, b):
    return jax.lax.dot_general(
        a.astype(jnp.bfloat16),
        b.astype(jnp.bfloat16),
        (((1,), (0,)), ((), ())),
        preferred_element_type=jnp.float32,
    )


def _prep_kernel(x_ref, w_ref, v_ref, xw_ref, s_ref):
    xw = _dot1(x_ref[...], w_ref[...])
    xw_ref[...] = xw
    f1 = _dot1(xw, v_ref[0])
    f2 = _dot1(xw, v_ref[1])
    s_ref[...] = jax.nn.sigmoid(f1 + f2)


def _half_softmax(p, s2):
    vals = p * s2
    e = jnp.where(vals != 0, jax.lax.exp2(vals), 0.0)
    tot = jnp.sum(e, axis=-1, keepdims=True)
    c = jnp.where(tot > 0, 0.5 / tot, 0.0)
    return e * c


def _mono_kernel(local_ref, long_ref, s_ref, xw_ref, wt_ref, out_ref,
                 att_scr, h_scr):
    i = pl.program_id(0)

    @pl.when(i < NA)
    def _softmax_phase():
        s2 = s_ref[...] * 1.4426950408889634
        att = _half_softmax(long_ref[...], s2) + _half_softmax(local_ref[...], s2)
        att16 = att.astype(jnp.bfloat16)
        att_scr[pl.ds(i * BRA, BRA), :] = att16
        h_scr[pl.ds(i * BRA, BRA), :] = jax.lax.dot_general(
            att16,
            xw_ref[...].astype(jnp.bfloat16),
            (((1,), (0,)), ((), ())),
            preferred_element_type=jnp.float32,
        )

    @pl.when(i == NA)
    def _bn_phase():
        h = h_scr[...]
        mu = jnp.mean(h, axis=0, keepdims=True)
        var = jnp.mean((h - mu) ** 2, axis=0, keepdims=True)
        h_scr[...] = (h - mu) / jnp.sqrt(var + 1e-6)

    @pl.when(i >= NA)
    def _decoder_phase():
        j = i - NA
        att16 = att_scr[pl.ds(j * BRB, BRB), :]
        dz = jax.lax.dot_general(
            att16,
            h_scr[...].astype(jnp.bfloat16),
            (((1,), (0,)), ((), ())),
            preferred_element_type=jnp.float32,
        )
        dz = _dot1(dz, wt_ref[...])
        out_ref[...] = jnp.where(dz > 0, dz, jnp.exp(jnp.minimum(dz, 0.0)) - 1.0)


@functools.partial(jax.jit, static_argnames=())
def kernel(local_patten, long_range_patten, X, W, V):
    xw, s = pl.pallas_call(
        _prep_kernel,
        out_shape=(
            jax.ShapeDtypeStruct((N, D_HID), jnp.float32),
            jax.ShapeDtypeStruct((N, 1), jnp.float32),
        ),
    )(X, W, V)

    out = pl.pallas_call(
        _mono_kernel,
        grid=(NA + NB,),
        in_specs=[
            pl.BlockSpec((BRA, N), lambda i: (jnp.minimum(i, NA - 1), 0)),
            pl.BlockSpec((BRA, N), lambda i: (jnp.minimum(i, NA - 1), 0)),
            pl.BlockSpec((BRA, 1), lambda i: (jnp.minimum(i, NA - 1), 0)),
            pl.BlockSpec((N, D_HID), lambda i: (0, 0)),
            pl.BlockSpec((D_HID, D_IN), lambda i: (0, 0)),
        ],
        out_specs=pl.BlockSpec(
            (BRB, D_IN), lambda i: (jnp.maximum(i - NA, 0), 0)
        ),
        out_shape=jax.ShapeDtypeStruct((N, D_IN), jnp.float32),
        scratch_shapes=[
            pltpu.VMEM((N, N), jnp.bfloat16),
            pltpu.VMEM((N, D_HID), jnp.float32),
        ],
        compiler_params=pltpu.CompilerParams(vmem_limit_bytes=67000000),
    )(local_patten, long_range_patten, s, xw, W.T)

    return out


# speedup vs baseline: 1.3127x; 1.0459x over previous
import functools

import jax
import jax.numpy as jnp
from jax.experimental import pallas as pl
from jax.experimental.pallas import tpu as pltpu

N = 4096
D_IN = 128
D_HID = 64
BRA = 256
BRB = 512
NA = N // BRA
NB = N // BRB


def _dot1(a, b):
    return jax.lax.dot_general(
        a.astype(jnp.bfloat16),
        b.astype(jnp.bfloat16),
        (((1,), (0,)), ((), ())),
        preferred_element_type=jnp.float32,
    )


def _half_softmax(p, s2):
    vals = p * s2
    e = jnp.where(vals != 0, jax.lax.exp2(vals), 0.0)
    tot = jnp.sum(e, axis=-1, keepdims=True)
    c = jnp.where(tot > 0, 0.5 / tot, 0.0)
    return e * c


def _mono_kernel(local_ref, long_ref, x_ref, w_ref, v_ref, wt_ref, out_ref,
                 att_scr, h_scr, xw_scr, s_scr):
    i = pl.program_id(0)

    @pl.when(i == 0)
    def _prep_phase():
        xw = _dot1(x_ref[...], w_ref[...])
        xw_scr[...] = xw.astype(jnp.bfloat16)
        f1 = _dot1(xw, v_ref[0])
        f2 = _dot1(xw, v_ref[1])
        s_scr[...] = jax.nn.sigmoid(f1 + f2) * 1.4426950408889634

    @pl.when(i < NA)
    def _softmax_phase():
        s2 = s_scr[pl.ds(i * BRA, BRA), :]
        att = _half_softmax(long_ref[...], s2) + _half_softmax(local_ref[...], s2)
        att16 = att.astype(jnp.bfloat16)
        att_scr[pl.ds(i * BRA, BRA), :] = att16
        h_scr[pl.ds(i * BRA, BRA), :] = jax.lax.dot_general(
            att16,
            xw_scr[...],
            (((1,), (0,)), ((), ())),
            preferred_element_type=jnp.float32,
        )

    @pl.when(i == NA)
    def _bn_phase():
        h = h_scr[...]
        mu = jnp.mean(h, axis=0, keepdims=True)
        var = jnp.mean((h - mu) ** 2, axis=0, keepdims=True)
        h_scr[...] = (h - mu) / jnp.sqrt(var + 1e-6)

    @pl.when(i >= NA)
    def _decoder_phase():
        j = i - NA
        att16 = att_scr[pl.ds(j * BRB, BRB), :]
        dz = jax.lax.dot_general(
            att16,
            h_scr[...].astype(jnp.bfloat16),
            (((1,), (0,)), ((), ())),
            preferred_element_type=jnp.float32,
        )
        dz = _dot1(dz, wt_ref[...])
        out_ref[...] = jnp.where(dz > 0, dz, jnp.exp(jnp.minimum(dz, 0.0)) - 1.0)


@functools.partial(jax.jit, static_argnames=())
def kernel(local_patten, long_range_patten, X, W, V):
    out = pl.pallas_call(
        _mono_kernel,
        grid=(NA + NB,),
        in_specs=[
            pl.BlockSpec((BRA, N), lambda i: (jnp.minimum(i, NA - 1), 0)),
            pl.BlockSpec((BRA, N), lambda i: (jnp.minimum(i, NA - 1), 0)),
            pl.BlockSpec((N, D_IN), lambda i: (0, 0)),
            pl.BlockSpec((D_IN, D_HID), lambda i: (0, 0)),
            pl.BlockSpec((2, D_HID, 1), lambda i: (0, 0, 0)),
            pl.BlockSpec((D_HID, D_IN), lambda i: (0, 0)),
        ],
        out_specs=pl.BlockSpec(
            (BRB, D_IN), lambda i: (jnp.maximum(i - NA, 0), 0)
        ),
        out_shape=jax.ShapeDtypeStruct((N, D_IN), jnp.float32),
        scratch_shapes=[
            pltpu.VMEM((N, N), jnp.bfloat16),
            pltpu.VMEM((N, D_HID), jnp.float32),
            pltpu.VMEM((N, D_HID), jnp.bfloat16),
            pltpu.VMEM((N, 1), jnp.float32),
        ],
        compiler_params=pltpu.CompilerParams(vmem_limit_bytes=67000000),
    )(local_patten, long_range_patten, X, W, V, W.T)

    return out


# speedup vs baseline: 1.3375x; 1.0189x over previous
import functools

import jax
import jax.numpy as jnp
from jax.experimental import pallas as pl
from jax.experimental.pallas import tpu as pltpu

N = 4096
D_IN = 128
D_HID = 64
BRA = 256
BRB = 1024
NA = N // BRA
NB = N // BRB


def _dot1(a, b):
    return jax.lax.dot_general(
        a.astype(jnp.bfloat16),
        b.astype(jnp.bfloat16),
        (((1,), (0,)), ((), ())),
        preferred_element_type=jnp.float32,
    )


def _half_softmax(p, s2):
    vals = p * s2
    e = jnp.where(vals != 0, jax.lax.exp2(vals), 0.0)
    tot = jnp.sum(e, axis=-1, keepdims=True)
    c = jnp.where(tot > 0, 0.5 / tot, 0.0)
    return e * c


def _mono_kernel(local_ref, long_ref, x_ref, w_ref, v_ref, wt_ref, out_ref,
                 att_scr, h_scr, xw_scr):
    i = pl.program_id(0)

    @pl.when(i == 0)
    def _prep_phase():
        xw_scr[...] = _dot1(x_ref[...], w_ref[...]).astype(jnp.bfloat16)

    @pl.when(i < NA)
    def _softmax_phase():
        xw_rows = xw_scr[pl.ds(i * BRA, BRA), :]
        f1 = _dot1(xw_rows, v_ref[0])
        f2 = _dot1(xw_rows, v_ref[1])
        s2 = jax.nn.sigmoid(f1 + f2) * 1.4426950408889634
        att = _half_softmax(long_ref[...], s2) + _half_softmax(local_ref[...], s2)
        att16 = att.astype(jnp.bfloat16)
        att_scr[pl.ds(i * BRA, BRA), :] = att16
        h_scr[pl.ds(i * BRA, BRA), :] = jax.lax.dot_general(
            att16,
            xw_scr[...],
            (((1,), (0,)), ((), ())),
            preferred_element_type=jnp.float32,
        )

    @pl.when(i == NA)
    def _bn_phase():
        h = h_scr[...]
        mu = jnp.mean(h, axis=0, keepdims=True)
        var = jnp.mean((h - mu) ** 2, axis=0, keepdims=True)
        h_scr[...] = (h - mu) / jnp.sqrt(var + 1e-6)

    @pl.when(i >= NA)
    def _decoder_phase():
        j = i - NA
        att16 = att_scr[pl.ds(j * BRB, BRB), :]
        dz = jax.lax.dot_general(
            att16,
            h_scr[...].astype(jnp.bfloat16),
            (((1,), (0,)), ((), ())),
            preferred_element_type=jnp.float32,
        )
        dz = _dot1(dz, wt_ref[...])
        out_ref[...] = jnp.where(dz > 0, dz, jnp.exp(jnp.minimum(dz, 0.0)) - 1.0)


@functools.partial(jax.jit, static_argnames=())
def kernel(local_patten, long_range_patten, X, W, V):
    out = pl.pallas_call(
        _mono_kernel,
        grid=(NA + NB,),
        in_specs=[
            pl.BlockSpec((BRA, N), lambda i: (jnp.minimum(i, NA - 1), 0)),
            pl.BlockSpec((BRA, N), lambda i: (jnp.minimum(i, NA - 1), 0)),
            pl.BlockSpec((N, D_IN), lambda i: (0, 0)),
            pl.BlockSpec((D_IN, D_HID), lambda i: (0, 0)),
            pl.BlockSpec((2, D_HID, 1), lambda i: (0, 0, 0)),
            pl.BlockSpec((D_HID, D_IN), lambda i: (0, 0)),
        ],
        out_specs=pl.BlockSpec(
            (BRB, D_IN), lambda i: (jnp.maximum(i - NA, 0), 0)
        ),
        out_shape=jax.ShapeDtypeStruct((N, D_IN), jnp.float32),
        scratch_shapes=[
            pltpu.VMEM((N, N), jnp.bfloat16),
            pltpu.VMEM((N, D_HID), jnp.float32),
            pltpu.VMEM((N, D_HID), jnp.bfloat16),
        ],
        compiler_params=pltpu.CompilerParams(vmem_limit_bytes=67000000),
    )(local_patten, long_range_patten, X, W, V, W.T)

    return out


# speedup vs baseline: 1.3455x; 1.0059x over previous
import functools

import jax
import jax.numpy as jnp
from jax.experimental import pallas as pl
from jax.experimental.pallas import tpu as pltpu

N = 4096
D_IN = 128
D_HID = 64
BRA = 256
BRB = 2048
NA = N // BRA
NB = N // BRB


def _dot1(a, b):
    return jax.lax.dot_general(
        a.astype(jnp.bfloat16),
        b.astype(jnp.bfloat16),
        (((1,), (0,)), ((), ())),
        preferred_element_type=jnp.float32,
    )


def _half_softmax(p, s2):
    vals = p * s2
    e = jnp.where(vals != 0, jax.lax.exp2(vals), 0.0)
    tot = jnp.sum(e, axis=-1, keepdims=True)
    c = jnp.where(tot > 0, 0.5 / tot, 0.0)
    return e * c


def _mono_kernel(local_ref, long_ref, x_ref, w_ref, v_ref, wt_ref, out_ref,
                 att_scr, h_scr, xw_scr):
    i = pl.program_id(0)

    @pl.when(i == 0)
    def _prep_phase():
        xw_scr[...] = _dot1(x_ref[...], w_ref[...]).astype(jnp.bfloat16)

    @pl.when(i < NA)
    def _softmax_phase():
        xw_rows = xw_scr[pl.ds(i * BRA, BRA), :]
        f1 = _dot1(xw_rows, v_ref[0])
        f2 = _dot1(xw_rows, v_ref[1])
        s2 = jax.nn.sigmoid(f1 + f2) * 1.4426950408889634
        att = _half_softmax(long_ref[...], s2) + _half_softmax(local_ref[...], s2)
        att16 = att.astype(jnp.bfloat16)
        att_scr[pl.ds(i * BRA, BRA), :] = att16
        h_scr[pl.ds(i * BRA, BRA), :] = jax.lax.dot_general(
            att16,
            xw_scr[...],
            (((1,), (0,)), ((), ())),
            preferred_element_type=jnp.float32,
        )

    @pl.when(i == NA)
    def _bn_phase():
        h = h_scr[...]
        mu = jnp.mean(h, axis=0, keepdims=True)
        var = jnp.mean((h - mu) ** 2, axis=0, keepdims=True)
        h_scr[...] = (h - mu) / jnp.sqrt(var + 1e-6)

    @pl.when(i >= NA)
    def _decoder_phase():
        j = i - NA
        att16 = att_scr[pl.ds(j * BRB, BRB), :]
        dz = jax.lax.dot_general(
            att16,
            h_scr[...].astype(jnp.bfloat16),
            (((1,), (0,)), ((), ())),
            preferred_element_type=jnp.float32,
        )
        dz = _dot1(dz, wt_ref[...])
        out_ref[...] = jnp.where(dz > 0, dz, jnp.exp(jnp.minimum(dz, 0.0)) - 1.0)


@functools.partial(jax.jit, static_argnames=())
def kernel(local_patten, long_range_patten, X, W, V):
    out = pl.pallas_call(
        _mono_kernel,
        grid=(NA + NB,),
        in_specs=[
            pl.BlockSpec((BRA, N), lambda i: (jnp.minimum(i, NA - 1), 0)),
            pl.BlockSpec((BRA, N), lambda i: (jnp.minimum(i, NA - 1), 0)),
            pl.BlockSpec((N, D_IN), lambda i: (0, 0)),
            pl.BlockSpec((D_IN, D_HID), lambda i: (0, 0)),
            pl.BlockSpec((2, D_HID, 1), lambda i: (0, 0, 0)),
            pl.BlockSpec((D_HID, D_IN), lambda i: (0, 0)),
        ],
        out_specs=pl.BlockSpec(
            (BRB, D_IN), lambda i: (jnp.maximum(i - NA, 0), 0)
        ),
        out_shape=jax.ShapeDtypeStruct((N, D_IN), jnp.float32),
        scratch_shapes=[
            pltpu.VMEM((N, N), jnp.bfloat16),
            pltpu.VMEM((N, D_HID), jnp.float32),
            pltpu.VMEM((N, D_HID), jnp.bfloat16),
        ],
        compiler_params=pltpu.CompilerParams(vmem_limit_bytes=67000000),
    )(local_patten, long_range_patten, X, W, V, W.T)

    return out
